# Initial kernel scaffold; baseline (speedup 1.0000x reference)
import jax
import jax.numpy as jnp
from jax.experimental import pallas as pl


def kernel(hidden_states, instrument_ids, session_ids, instrument_table, session_table):
    raise NotImplementedError("write your pallas kernel here")


# speedup vs baseline: 2.0976x; 2.0976x over previous
import functools

import jax
import jax.numpy as jnp
from jax import lax
from jax.experimental import pallas as pl
from jax.experimental.pallas import tpu as pltpu
from jax.experimental.pallas import tpu_sc as plsc

B = 4096
S = 200
H = 64
HALF = S // 2


def _make_kernel():
    info = plsc.get_sparse_core_info()
    nc, ns = info.num_cores, info.num_subcores
    nw = nc * ns
    b_per_w = B // nw

    mesh = plsc.VectorSubcoreMesh(core_axis_name="c", subcore_axis_name="s")

    @functools.partial(
        pl.kernel,
        mesh=mesh,
        out_type=jax.ShapeDtypeStruct((B, S, H), jnp.float32),
        compiler_params=pltpu.CompilerParams(use_tc_tiling_on_sc=False),
        scratch_types=[
            pltpu.VMEM((b_per_w,), jnp.int32),
            pltpu.VMEM((b_per_w, H), jnp.float32),
            pltpu.VMEM((2, HALF), jnp.int32),
            pltpu.VMEM((S, H), jnp.float32),
            pltpu.VMEM((S, H), jnp.float32),
            pltpu.SemaphoreType.DMA,
            pltpu.SemaphoreType.DMA,
            pltpu.SemaphoreType.DMA,
        ],
    )
    def k(hid_hbm, iids_hbm, sids_hbm, itab_hbm, stab_hbm, out_hbm,
          iid_v, irows_v, sid_v, hid_v, srows_v, sem0, sem1, sem2):
        wid = lax.axis_index("s") * nc + lax.axis_index("c")
        b0 = wid * b_per_w

        pltpu.sync_copy(iids_hbm.at[pl.ds(b0, b_per_w)], iid_v)
        pltpu.async_copy(itab_hbm.at[iid_v], irows_v, sem0).wait()

        def batch_body(m, _):
            b = b0 + m
            cp_h = pltpu.async_copy(hid_hbm.at[b], hid_v, sem0)
            pltpu.sync_copy(sids_hbm.at[b], sid_v)
            cp_s0 = pltpu.async_copy(
                stab_hbm.at[sid_v.at[0]], srows_v.at[pl.ds(0, HALF)], sem1)
            cp_s1 = pltpu.async_copy(
                stab_hbm.at[sid_v.at[1]], srows_v.at[pl.ds(HALF, HALF)], sem2)

            iv0 = irows_v[m, pl.ds(0, 16)]
            iv1 = irows_v[m, pl.ds(16, 16)]
            iv2 = irows_v[m, pl.ds(32, 16)]
            iv3 = irows_v[m, pl.ds(48, 16)]

            cp_h.wait()
            cp_s0.wait()
            cp_s1.wait()

            def row_body(r, _):
                hid_v[r, pl.ds(0, 16)] = (
                    hid_v[r, pl.ds(0, 16)] + srows_v[r, pl.ds(0, 16)] + iv0)
                hid_v[r, pl.ds(16, 16)] = (
                    hid_v[r, pl.ds(16, 16)] + srows_v[r, pl.ds(16, 16)] + iv1)
                hid_v[r, pl.ds(32, 16)] = (
                    hid_v[r, pl.ds(32, 16)] + srows_v[r, pl.ds(32, 16)] + iv2)
                hid_v[r, pl.ds(48, 16)] = (
                    hid_v[r, pl.ds(48, 16)] + srows_v[r, pl.ds(48, 16)] + iv3)
                return 0

            lax.fori_loop(0, S, row_body, 0)
            pltpu.async_copy(hid_v, out_hbm.at[b], sem0).wait()
            return 0

        lax.fori_loop(0, b_per_w, batch_body, 0)

    return k


_kernel_call = None


def kernel(hidden_states, instrument_ids, session_ids, instrument_table, session_table):
    global _kernel_call
    if _kernel_call is None:
        _kernel_call = _make_kernel()
    sids = session_ids.reshape(B, 2, HALF).astype(jnp.int32)
    iids = instrument_ids.astype(jnp.int32)
    return _kernel_call(hidden_states, iids, sids, instrument_table, session_table)


# speedup vs baseline: 2.2699x; 1.0821x over previous
import functools

import jax
import jax.numpy as jnp
from jax import lax
from jax.experimental import pallas as pl
from jax.experimental.pallas import tpu as pltpu
from jax.experimental.pallas import tpu_sc as plsc

B = 4096
S = 200
H = 64
HALF = S // 2


def _make_kernel():
    info = plsc.get_sparse_core_info()
    nc, ns = info.num_cores, info.num_subcores
    nw = nc * ns
    b_per_w = B // nw

    mesh = plsc.VectorSubcoreMesh(core_axis_name="c", subcore_axis_name="s")

    @functools.partial(
        pl.kernel,
        mesh=mesh,
        out_type=jax.ShapeDtypeStruct((B, S, H), jnp.float32),
        compiler_params=pltpu.CompilerParams(use_tc_tiling_on_sc=False),
        scratch_types=[
            pltpu.VMEM((b_per_w,), jnp.int32),
            pltpu.VMEM((b_per_w, H), jnp.float32),
            pltpu.VMEM((b_per_w, 2, HALF), jnp.int32),
            pltpu.VMEM((S, H), jnp.float32),
            pltpu.VMEM((S, H), jnp.float32),
            pltpu.VMEM((S, H), jnp.float32),
            pltpu.VMEM((S, H), jnp.float32),
            pltpu.SemaphoreType.DMA,
            pltpu.SemaphoreType.DMA,
            pltpu.SemaphoreType.DMA,
            pltpu.SemaphoreType.DMA,
            pltpu.SemaphoreType.DMA,
            pltpu.SemaphoreType.DMA,
        ],
    )
    def k(hid_hbm, iids_hbm, sids_hbm, itab_hbm, stab_hbm, out_hbm,
          iid_v, irows_v, sid_v, hid0, hid1, srows0, srows1,
          hsem0, hsem1, gsem0, gsem1, osem0, osem1):
        wid = lax.axis_index("s") * nc + lax.axis_index("c")
        b0 = wid * b_per_w

        bufs = ((hid0, srows0, hsem0, gsem0, osem0),
                (hid1, srows1, hsem1, gsem1, osem1))

        pltpu.sync_copy(iids_hbm.at[pl.ds(b0, b_per_w)], iid_v)
        pltpu.sync_copy(sids_hbm.at[pl.ds(b0, b_per_w)], sid_v)
        pltpu.async_copy(itab_hbm.at[iid_v], irows_v, gsem0).wait()

        def issue_in(m, hid, srows, hsem, gsem):
            pltpu.async_copy(hid_hbm.at[b0 + m], hid, hsem)
            pltpu.async_copy(
                stab_hbm.at[sid_v.at[m, 0]], srows.at[pl.ds(0, HALF)], gsem)
            pltpu.async_copy(
                stab_hbm.at[sid_v.at[m, 1]], srows.at[pl.ds(HALF, HALF)], gsem)

        def wait_in(hid, srows, hsem, gsem):
            pltpu.make_async_copy(hid_hbm.at[0], hid, hsem).wait()
            pltpu.make_async_copy(
                stab_hbm.at[pl.ds(0, HALF)], srows.at[pl.ds(0, HALF)], gsem).wait()
            pltpu.make_async_copy(
                stab_hbm.at[pl.ds(0, HALF)], srows.at[pl.ds(HALF, HALF)], gsem).wait()

        issue_in(0, hid0, srows0, hsem0, gsem0)

        def pair_body(g, _):
            for j in (0, 1):
                hid, srows, hsem, gsem, osem = bufs[j]
                nhid, nsrows, nhsem, ngsem, nosem = bufs[1 - j]
                cur = 2 * g + j

                @pl.when(cur >= 1)
                def _():
                    pltpu.make_async_copy(
                        nhid, out_hbm.at[b0], nosem).wait()

                @pl.when(cur + 1 < b_per_w)
                def _():
                    issue_in(cur + 1, nhid, nsrows, nhsem, ngsem)

                wait_in(hid, srows, hsem, gsem)

                iv0 = irows_v[cur, pl.ds(0, 16)]
                iv1 = irows_v[cur, pl.ds(16, 16)]
                iv2 = irows_v[cur, pl.ds(32, 16)]
                iv3 = irows_v[cur, pl.ds(48, 16)]

                def row_body(r, _):
                    hid[r, pl.ds(0, 16)] = (
                        hid[r, pl.ds(0, 16)] + srows[r, pl.ds(0, 16)] + iv0)
                    hid[r, pl.ds(16, 16)] = (
                        hid[r, pl.ds(16, 16)] + srows[r, pl.ds(16, 16)] + iv1)
                    hid[r, pl.ds(32, 16)] = (
                        hid[r, pl.ds(32, 16)] + srows[r, pl.ds(32, 16)] + iv2)
                    hid[r, pl.ds(48, 16)] = (
                        hid[r, pl.ds(48, 16)] + srows[r, pl.ds(48, 16)] + iv3)
                    return 0

                lax.fori_loop(0, S, row_body, 0)
                pltpu.async_copy(hid, out_hbm.at[b0 + cur], osem)
            return 0

        lax.fori_loop(0, b_per_w // 2, pair_body, 0)
        pltpu.make_async_copy(hid1, out_hbm.at[b0], osem1).wait()

    return k


_kernel_call = None


def kernel(hidden_states, instrument_ids, session_ids, instrument_table, session_table):
    global _kernel_call
    if _kernel_call is None:
        _kernel_call = _make_kernel()
    sids = session_ids.reshape(B, 2, HALF).astype(jnp.int32)
    iids = instrument_ids.astype(jnp.int32)
    return _kernel_call(hidden_states, iids, sids, instrument_table, session_table)
